# Initial kernel scaffold; baseline (speedup 1.0000x reference)
import jax
import jax.numpy as jnp
from jax.experimental import pallas as pl


def kernel(expert_outputs_TED, weights_TE):
    raise NotImplementedError("write your pallas kernel here")


# speedup vs baseline: 2.7076x; 2.7076x over previous
import jax
import jax.numpy as jnp
from jax.experimental import pallas as pl


def _combine_body(x_ref, w_ref, o_ref):
    E = x_ref.shape[1]
    acc = x_ref[:, 0, :] * w_ref[:, 0:1]
    for e in range(1, E):
        acc += x_ref[:, e, :] * w_ref[:, e : e + 1]
    o_ref[...] = acc


def kernel(expert_outputs_TED, weights_TE):
    T, E, D = expert_outputs_TED.shape
    BT = 256
    return pl.pallas_call(
        _combine_body,
        grid=(T // BT,),
        in_specs=[
            pl.BlockSpec((BT, E, D), lambda i: (i, 0, 0)),
            pl.BlockSpec((BT, E), lambda i: (i, 0)),
        ],
        out_specs=pl.BlockSpec((BT, D), lambda i: (i, 0)),
        out_shape=jax.ShapeDtypeStruct((T, D), jnp.float32),
    )(expert_outputs_TED, weights_TE)
